# baseline (speedup 1.0000x reference)
import functools

import jax
import jax.numpy as jnp
from jax import lax
from jax.experimental import pallas as pl
from jax.experimental.pallas import tpu as pltpu
from jax.experimental.pallas import tpu_sc as plsc

N = 10000
E = 160000
V = 100000
G = 4096
FIN = 20
HDIM = 1024

NPAD = 10240
EPAD = 163840
NC, NS, NW = 2, 16, 32
CHUNKS = 8
CPC = CHUNKS // NC
RPT = NPAD // NS
F32 = jnp.float32

_MESH = plsc.VectorSubcoreMesh(
    core_axis_name="c", subcore_axis_name="s", num_cores=NC, num_subcores=NS)


def _edge_scan(table, idx_s, idx_d, bufa, bufb, gsa, gsb, ssa, ssb, acc,
               ngroups):
  npair = ngroups // 2

  def _g(j, buf, sem):
    pltpu.async_copy(table.at[idx_s.at[j]], buf, sem)

  def _gwait(buf, sem):
    pltpu.make_async_copy(table.at[idx_s.at[0]], buf, sem).wait()

  def _s(j, buf, sem):
    pltpu.async_copy(buf, acc.at[idx_d.at[j]], sem, add=True)

  def _swait(buf, sem):
    pltpu.make_async_copy(buf, acc.at[idx_d.at[0]], sem).wait()

  _g(0, bufa, gsa)
  _g(1, bufb, gsb)

  def body(t, carry):
    _gwait(bufa, gsa)
    _s(2 * t, bufa, ssa)
    _gwait(bufb, gsb)
    _s(2 * t + 1, bufb, ssb)
    _swait(bufa, ssa)
    _g(2 * t + 2, bufa, gsa)
    _swait(bufb, ssb)
    _g(2 * t + 3, bufb, gsb)
    return carry

  lax.fori_loop(0, npair - 1, body, 0)
  _gwait(bufa, gsa)
  _s(ngroups - 2, bufa, ssa)
  _gwait(bufb, gsb)
  _s(ngroups - 1, bufb, ssb)
  _swait(bufa, ssa)
  _swait(bufb, ssb)


def _sc_conv1(esrc1, edst1, hpad, z128):
  @functools.partial(
      pl.kernel,
      out_type=jax.ShapeDtypeStruct((NC, NPAD, 128), F32),
      mesh=_MESH,
      scratch_types=[
          pltpu.VMEM((40, 128), jnp.int32),
          pltpu.VMEM((40, 128), jnp.int32),
          pltpu.VMEM((128, 128), F32),
          pltpu.VMEM((128, 128), F32),
          pltpu.VMEM_SHARED((NPAD, 128), F32),
          pltpu.SemaphoreType.DMA,
          pltpu.SemaphoreType.DMA,
          pltpu.SemaphoreType.DMA,
          pltpu.SemaphoreType.DMA,
      ],
  )
  def body(esrc_r, edst_r, hpad_r, z_r, s1_out,
           idx_s, idx_d, bufa, bufb, acc, gsa, gsb, ssa, ssb):
    cid = lax.axis_index("c")
    sid = lax.axis_index("s")
    wid = sid * NC + cid
    r0 = sid * RPT
    pltpu.sync_copy(z_r.at[pl.ds(r0, RPT)], acc.at[pl.ds(r0, RPT)])
    pltpu.sync_copy(esrc_r.at[wid], idx_s)
    pltpu.sync_copy(edst_r.at[wid], idx_d)
    plsc.subcore_barrier()
    _edge_scan(hpad_r, idx_s, idx_d, bufa, bufb, gsa, gsb, ssa, ssb, acc, 40)
    plsc.subcore_barrier()
    pltpu.sync_copy(acc.at[pl.ds(r0, RPT)], s1_out.at[cid, pl.ds(r0, RPT)])

  return body(esrc1, edst1, hpad, z128)


def _sc_conv2(esrc2, edst2, x2f, z128):
  @functools.partial(
      pl.kernel,
      out_type=jax.ShapeDtypeStruct((CHUNKS * NPAD, 128), F32),
      mesh=_MESH,
      scratch_types=[
          pltpu.VMEM((40, 128), jnp.int32),
          pltpu.VMEM((40, 128), jnp.int32),
          pltpu.VMEM((128, 128), F32),
          pltpu.VMEM((128, 128), F32),
          pltpu.VMEM_SHARED((NPAD, 128), F32),
          pltpu.SemaphoreType.DMA,
          pltpu.SemaphoreType.DMA,
          pltpu.SemaphoreType.DMA,
          pltpu.SemaphoreType.DMA,
      ],
  )
  def body(esrc_r, edst_r, x2f_r, z_r, s2_out,
           idx_s, idx_d, bufa, bufb, acc, gsa, gsb, ssa, ssb):
    cid = lax.axis_index("c")
    sid = lax.axis_index("s")
    r0 = sid * RPT
    for k in range(CPC):
      pltpu.sync_copy(z_r.at[pl.ds(r0, RPT)], acc.at[pl.ds(r0, RPT)])
      plsc.subcore_barrier()
      for half in range(2):
        pltpu.sync_copy(esrc_r.at[cid, k, sid, pl.ds(half * 40, 40)], idx_s)
        pltpu.sync_copy(edst_r.at[sid, pl.ds(half * 40, 40)], idx_d)
        _edge_scan(x2f_r, idx_s, idx_d, bufa, bufb, gsa, gsb, ssa, ssb,
                   acc, 40)
      plsc.subcore_barrier()
      off = (cid * CPC + k) * NPAD + r0
      pltpu.sync_copy(acc.at[pl.ds(r0, RPT)], s2_out.at[pl.ds(off, RPT)])

  return body(esrc2, edst2, x2f, z128)


def _sc_gather(table, gpos):
  rows_per_w = G // NW

  @functools.partial(
      pl.kernel,
      out_type=jax.ShapeDtypeStruct((G, HDIM), F32),
      mesh=_MESH,
      scratch_types=[
          pltpu.VMEM((rows_per_w,), jnp.int32),
          pltpu.VMEM((rows_per_w // 2, HDIM), F32),
          pltpu.SemaphoreType.DMA,
      ],
  )
  def body(table_r, gpos_r, out_r, idx_v, rows, sem):
    cid = lax.axis_index("c")
    sid = lax.axis_index("s")
    wid = sid * NC + cid
    pltpu.sync_copy(gpos_r.at[pl.ds(wid * rows_per_w, rows_per_w)], idx_v)
    for half in range(2):
      hh = rows_per_w // 2
      pltpu.async_copy(
          table_r.at[idx_v.at[pl.ds(half * hh, hh)]], rows, sem).wait()
      pltpu.sync_copy(rows, out_r.at[pl.ds(wid * rows_per_w + half * hh, hh)])

  return body(table, gpos)


def _tc_fc_bn1(hpad, s1p, Wp, b2, g2, bb2):
  NB, BR = 8, NPAD // 8

  def body(h_ref, s1_ref, w_ref, b_ref, g_ref, bb_ref, o_ref,
           ssum, ssq):
    p = pl.program_id(0)
    i = pl.program_id(1)
    s1 = s1_ref[0] + s1_ref[1]
    cnt = s1[:, 20][:, None]
    u = h_ref[...] + s1 / jnp.maximum(cnt, 1.0)
    z = jnp.dot(u, w_ref[...], preferred_element_type=F32)
    z = z + b_ref[...] * (1.0 + (cnt > 0).astype(F32))
    rid = i * BR + lax.broadcasted_iota(jnp.int32, (BR, 1), 0)
    msk = (rid < N).astype(F32)

    @pl.when(jnp.logical_and(p == 0, i == 0))
    def _():
      ssum[...] = jnp.zeros_like(ssum)
      ssq[...] = jnp.zeros_like(ssq)

    @pl.when(p == 0)
    def _():
      zm = z * msk
      ssum[...] += jnp.sum(zm, 0, keepdims=True)
      ssq[...] += jnp.sum(zm * z, 0, keepdims=True)

    @pl.when(p == 1)
    def _():
      mean = ssum[...] / N
      var = ssq[...] / N - mean * mean
      inv = lax.rsqrt(var + 1e-5)
      y = jnp.maximum((z - mean) * inv * g_ref[...] + bb_ref[...], 0.0)
      for k in range(CHUNKS):
        o_ref[k] = y[:, k * 128:(k + 1) * 128]

  return pl.pallas_call(
      body,
      grid=(2, NB),
      in_specs=[
          pl.BlockSpec((BR, 128), lambda p, i: (i, 0)),
          pl.BlockSpec((NC, BR, 128), lambda p, i: (0, i, 0)),
          pl.BlockSpec((128, HDIM), lambda p, i: (0, 0)),
          pl.BlockSpec((1, HDIM), lambda p, i: (0, 0)),
          pl.BlockSpec((1, HDIM), lambda p, i: (0, 0)),
          pl.BlockSpec((1, HDIM), lambda p, i: (0, 0)),
      ],
      out_specs=pl.BlockSpec((CHUNKS, BR, 128), lambda p, i: (0, i, 0)),
      out_shape=jax.ShapeDtypeStruct((CHUNKS, NPAD, 128), F32),
      scratch_shapes=[pltpu.VMEM((1, HDIM), F32), pltpu.VMEM((1, HDIM), F32)],
  )(hpad, s1p, Wp, b2, g2, bb2)


def _tc_bn2_qemb(x2r, s2r, s1p, g2, bb2):
  NB, BR = 8, NPAD // 8

  def body(x_ref, s_ref, c_ref, g_ref, bb_ref, q_ref, ssum, ssq, qacc):
    p = pl.program_id(0)
    i = pl.program_id(1)
    cnt = (c_ref[0, :, 20] + c_ref[1, :, 20])[:, None]
    rden = 1.0 / jnp.maximum(cnt, 1.0)
    rid = i * BR + lax.broadcasted_iota(jnp.int32, (BR, 1), 0)
    msk = (rid < N).astype(F32)

    @pl.when(jnp.logical_and(p == 0, i == 0))
    def _():
      ssum[...] = jnp.zeros_like(ssum)
      ssq[...] = jnp.zeros_like(ssq)
      qacc[...] = jnp.zeros_like(qacc)

    @pl.when(p == 0)
    def _():
      for k in range(CHUNKS):
        sl = pl.ds(k * 128, 128)
        z = x_ref[k] + s_ref[k] * rden
        zm = z * msk
        ssum[0, sl] += jnp.sum(zm, 0)
        ssq[0, sl] += jnp.sum(zm * z, 0)

    @pl.when(p == 1)
    def _():
      for k in range(CHUNKS):
        sl = pl.ds(k * 128, 128)
        z = x_ref[k] + s_ref[k] * rden
        mean = ssum[0, sl] / N
        var = ssq[0, sl] / N - mean * mean
        inv = lax.rsqrt(var + 1e-5)
        y = jnp.maximum((z - mean) * inv * g_ref[0, sl] + bb_ref[0, sl], 0.0)
        qacc[0, sl] += jnp.sum(y * msk, 0)

      @pl.when(i == NB - 1)
      def _():
        q_ref[...] = qacc[...] / N

  return pl.pallas_call(
      body,
      grid=(2, NB),
      in_specs=[
          pl.BlockSpec((CHUNKS, BR, 128), lambda p, i: (0, i, 0)),
          pl.BlockSpec((CHUNKS, BR, 128), lambda p, i: (0, i, 0)),
          pl.BlockSpec((NC, BR, 128), lambda p, i: (0, i, 0)),
          pl.BlockSpec((1, HDIM), lambda p, i: (0, 0)),
          pl.BlockSpec((1, HDIM), lambda p, i: (0, 0)),
      ],
      out_specs=pl.BlockSpec((1, HDIM), lambda p, i: (0, 0)),
      out_shape=jax.ShapeDtypeStruct((1, HDIM), F32),
      scratch_shapes=[pltpu.VMEM((1, HDIM), F32), pltpu.VMEM((1, HDIM), F32),
                      pltpu.VMEM((1, HDIM), F32)],
  )(x2r, s2r, s1p, g2, bb2)


def _tc_cand(gemb, qemb, Wtop, Wbot, bfc2, g3, b3, w2row, bb2sc):
  NB, BR = 8, G // 8

  def body(ge_ref, q_ref, wt_ref, wb_ref, bf_ref, g_ref, bb_ref, w2_ref,
           b2_ref, o_ref, yscr, ssum, ssq):
    p = pl.program_id(0)
    i = pl.program_id(1)

    @pl.when(jnp.logical_and(p == 0, i == 0))
    def _():
      ssum[...] = jnp.zeros_like(ssum)
      ssq[...] = jnp.zeros_like(ssq)

    @pl.when(p == 0)
    def _():
      c0 = jnp.dot(q_ref[...], wt_ref[...], preferred_element_type=F32)
      yb = jnp.dot(ge_ref[...], wb_ref[...], preferred_element_type=F32)
      yb = yb + c0 + bf_ref[...]
      yscr[pl.ds(i * BR, BR), :] = yb
      ssum[...] += jnp.sum(yb, 0, keepdims=True)
      ssq[...] += jnp.sum(yb * yb, 0, keepdims=True)

    @pl.when(p == 1)
    def _():
      mean = ssum[...] / G
      var = ssq[...] / G - mean * mean
      inv = lax.rsqrt(var + 1e-5)
      yb = yscr[pl.ds(i * BR, BR), :]
      y = jnp.maximum((yb - mean) * inv * g_ref[...] + bb_ref[...], 0.0)
      logit = jnp.sum(y * w2_ref[...], axis=1, keepdims=True) + b2_ref[...]
      prob = 1.0 / (1.0 + jnp.exp(-logit))
      o_ref[...] = prob

  return pl.pallas_call(
      body,
      grid=(2, NB),
      in_specs=[
          pl.BlockSpec((BR, HDIM), lambda p, i: (i, 0)),
          pl.BlockSpec((1, HDIM), lambda p, i: (0, 0)),
          pl.BlockSpec((HDIM, 128), lambda p, i: (0, 0)),
          pl.BlockSpec((HDIM, 128), lambda p, i: (0, 0)),
          pl.BlockSpec((1, 128), lambda p, i: (0, 0)),
          pl.BlockSpec((1, 128), lambda p, i: (0, 0)),
          pl.BlockSpec((1, 128), lambda p, i: (0, 0)),
          pl.BlockSpec((1, 128), lambda p, i: (0, 0)),
          pl.BlockSpec((1, 1), lambda p, i: (0, 0)),
      ],
      out_specs=pl.BlockSpec((BR, 1), lambda p, i: (i, 0)),
      out_shape=jax.ShapeDtypeStruct((G, 1), F32),
      scratch_shapes=[pltpu.VMEM((G, 128), F32), pltpu.VMEM((1, 128), F32),
                      pltpu.VMEM((1, 128), F32)],
  )(gemb, qemb, Wtop, Wbot, bfc2, g3, b3, w2row, bb2sc)


def kernel(h, edge_index, allDBGEmb, gPos,
           W_init, b_init, bn1_scale, bn1_bias, bn2_scale, bn2_bias,
           W_fc, b_fc, bn3_scale, bn3_bias, W_fc2, b_fc2):
  i32 = jnp.int32
  hpad = (jnp.zeros((NPAD, 128), F32).at[:N, :FIN].set(h)
          .at[:, FIN].set(1.0))
  Wp = jnp.zeros((128, HDIM), F32).at[:FIN].set(W_init)
  epad = jnp.concatenate(
      [edge_index.astype(i32),
       jnp.full((2, EPAD - E), NPAD - 1, dtype=i32)], axis=1)
  src, dst = epad[0], epad[1]
  esrc1 = src.reshape(NW, 40, 128)
  edst1 = dst.reshape(NW, 40, 128)
  src_t = src.reshape(NS, 80, 128)
  edst2 = dst.reshape(NS, 80, 128)
  offs = (jnp.arange(CHUNKS, dtype=i32) * NPAD).reshape(NC, CPC, 1, 1, 1)
  esrc2 = src_t[None, None] + offs
  z128 = jnp.zeros((NPAD, 128), F32)

  gemb = _sc_gather(allDBGEmb, gPos)
  s1p = _sc_conv1(esrc1, edst1, hpad, z128)
  x2r = _tc_fc_bn1(hpad, s1p, Wp,
                   b_init.reshape(1, HDIM), bn1_scale.reshape(1, HDIM),
                   bn1_bias.reshape(1, HDIM))
  s2f = _sc_conv2(esrc2, edst2, x2r.reshape(CHUNKS * NPAD, 128), z128)
  qemb = _tc_bn2_qemb(x2r, s2f.reshape(CHUNKS, NPAD, 128), s1p,
                      bn2_scale.reshape(1, HDIM), bn2_bias.reshape(1, HDIM))
  probs = _tc_cand(gemb, qemb,
                   W_fc[:HDIM], W_fc[HDIM:], b_fc.reshape(1, 128),
                   bn3_scale.reshape(1, 128), bn3_bias.reshape(1, 128),
                   W_fc2.reshape(1, 128), b_fc2.reshape(1, 1))
  return probs.reshape(-1)


# speedup vs baseline: 2.4365x; 2.4365x over previous
import functools

import jax
import jax.numpy as jnp
from jax import lax
from jax.experimental import pallas as pl
from jax.experimental.pallas import tpu as pltpu
from jax.experimental.pallas import tpu_sc as plsc

N = 10000
E = 160000
V = 100000
G = 4096
FIN = 20
HDIM = 1024

NPAD = 10240
EPAD = 163840
NC, NS, NW = 2, 16, 32
CHUNKS = 8
CPC = CHUNKS // NC
RPT = NPAD // NS
F32 = jnp.float32

_MESH = plsc.VectorSubcoreMesh(
    core_axis_name="c", subcore_axis_name="s", num_cores=NC, num_subcores=NS)


def _edge_scan(table, idx_s, idx_d, bufa, bufb, sema, semb, acc, ngroups):
  npair = ngroups // 2

  def _scat(buf, j):
    pltpu.sync_copy(buf, acc.at[idx_d.at[j]], add=True)

  pltpu.async_copy(table.at[idx_s.at[0]], bufa, sema)

  def body(t, carry):
    pltpu.async_copy(table.at[idx_s.at[2 * t + 1]], bufb, semb)
    pltpu.make_async_copy(table.at[idx_s.at[0]], bufa, sema).wait()
    _scat(bufa, 2 * t)
    pltpu.async_copy(table.at[idx_s.at[2 * t + 2]], bufa, sema)
    pltpu.make_async_copy(table.at[idx_s.at[0]], bufb, semb).wait()
    _scat(bufb, 2 * t + 1)
    return carry

  lax.fori_loop(0, npair - 1, body, 0)
  pltpu.async_copy(table.at[idx_s.at[ngroups - 1]], bufb, semb)
  pltpu.make_async_copy(table.at[idx_s.at[0]], bufa, sema).wait()
  _scat(bufa, ngroups - 2)
  pltpu.make_async_copy(table.at[idx_s.at[0]], bufb, semb).wait()
  _scat(bufb, ngroups - 1)


def _sc_conv1(esrc1, edst1, hpad, z128):
  @functools.partial(
      pl.kernel,
      out_type=jax.ShapeDtypeStruct((NC, NPAD, 128), F32),
      mesh=_MESH,
      scratch_types=[
          pltpu.VMEM((40, 128), jnp.int32),
          pltpu.VMEM((40, 128), jnp.int32),
          pltpu.VMEM((128, 128), F32),
          pltpu.VMEM((128, 128), F32),
          pltpu.VMEM_SHARED((NPAD, 128), F32),
          pltpu.SemaphoreType.DMA,
          pltpu.SemaphoreType.DMA,
      ],
  )
  def body(esrc_r, edst_r, hpad_r, z_r, s1_out,
           idx_s, idx_d, bufa, bufb, acc, sema, semb):
    cid = lax.axis_index("c")
    sid = lax.axis_index("s")
    wid = sid * NC + cid
    r0 = sid * RPT
    pltpu.sync_copy(z_r.at[pl.ds(r0, RPT)], acc.at[pl.ds(r0, RPT)])
    pltpu.sync_copy(esrc_r.at[wid], idx_s)
    pltpu.sync_copy(edst_r.at[wid], idx_d)
    plsc.subcore_barrier()
    _edge_scan(hpad_r, idx_s, idx_d, bufa, bufb, sema, semb, acc, 40)
    plsc.subcore_barrier()
    pltpu.sync_copy(acc.at[pl.ds(r0, RPT)], s1_out.at[cid, pl.ds(r0, RPT)])

  return body(esrc1, edst1, hpad, z128)


def _sc_conv2(esrc2, edst2, x2f, z128):
  @functools.partial(
      pl.kernel,
      out_type=jax.ShapeDtypeStruct((CHUNKS * NPAD, 128), F32),
      mesh=_MESH,
      scratch_types=[
          pltpu.VMEM((40, 128), jnp.int32),
          pltpu.VMEM((40, 128), jnp.int32),
          pltpu.VMEM((128, 128), F32),
          pltpu.VMEM((128, 128), F32),
          pltpu.VMEM_SHARED((NPAD, 128), F32),
          pltpu.SemaphoreType.DMA,
          pltpu.SemaphoreType.DMA,
      ],
  )
  def body(esrc_r, edst_r, x2f_r, z_r, s2_out,
           idx_s, idx_d, bufa, bufb, acc, sema, semb):
    cid = lax.axis_index("c")
    sid = lax.axis_index("s")
    r0 = sid * RPT
    for k in range(CPC):
      pltpu.sync_copy(z_r.at[pl.ds(r0, RPT)], acc.at[pl.ds(r0, RPT)])
      plsc.subcore_barrier()
      for half in range(2):
        pltpu.sync_copy(esrc_r.at[cid, k, sid, pl.ds(half * 40, 40)], idx_s)
        pltpu.sync_copy(edst_r.at[sid, pl.ds(half * 40, 40)], idx_d)
        _edge_scan(x2f_r, idx_s, idx_d, bufa, bufb, sema, semb, acc, 40)
      plsc.subcore_barrier()
      off = (cid * CPC + k) * NPAD + r0
      pltpu.sync_copy(acc.at[pl.ds(r0, RPT)], s2_out.at[pl.ds(off, RPT)])

  return body(esrc2, edst2, x2f, z128)


def _sc_gather(table, gpos):
  rows_per_w = G // NW

  @functools.partial(
      pl.kernel,
      out_type=jax.ShapeDtypeStruct((G, HDIM), F32),
      mesh=_MESH,
      scratch_types=[
          pltpu.VMEM((rows_per_w,), jnp.int32),
          pltpu.VMEM((rows_per_w // 2, HDIM), F32),
          pltpu.SemaphoreType.DMA,
      ],
  )
  def body(table_r, gpos_r, out_r, idx_v, rows, sem):
    cid = lax.axis_index("c")
    sid = lax.axis_index("s")
    wid = sid * NC + cid
    pltpu.sync_copy(gpos_r.at[pl.ds(wid * rows_per_w, rows_per_w)], idx_v)
    for half in range(2):
      hh = rows_per_w // 2
      pltpu.async_copy(
          table_r.at[idx_v.at[pl.ds(half * hh, hh)]], rows, sem).wait()
      pltpu.sync_copy(rows, out_r.at[pl.ds(wid * rows_per_w + half * hh, hh)])

  return body(table, gpos)


def _tc_fc_bn1(hpad, s1p, Wp, b2, g2, bb2):
  NB, BR = 8, NPAD // 8

  def body(h_ref, s1_ref, w_ref, b_ref, g_ref, bb_ref, o_ref,
           ssum, ssq):
    p = pl.program_id(0)
    i = pl.program_id(1)
    s1 = s1_ref[0] + s1_ref[1]
    cnt = s1[:, 20][:, None]
    u = h_ref[...] + s1 / jnp.maximum(cnt, 1.0)
    z = jnp.dot(u, w_ref[...], preferred_element_type=F32)
    z = z + b_ref[...] * (1.0 + (cnt > 0).astype(F32))
    rid = i * BR + lax.broadcasted_iota(jnp.int32, (BR, 1), 0)
    msk = (rid < N).astype(F32)

    @pl.when(jnp.logical_and(p == 0, i == 0))
    def _():
      ssum[...] = jnp.zeros_like(ssum)
      ssq[...] = jnp.zeros_like(ssq)

    @pl.when(p == 0)
    def _():
      zm = z * msk
      ssum[...] += jnp.sum(zm, 0, keepdims=True)
      ssq[...] += jnp.sum(zm * z, 0, keepdims=True)

    @pl.when(p == 1)
    def _():
      mean = ssum[...] / N
      var = ssq[...] / N - mean * mean
      inv = lax.rsqrt(var + 1e-5)
      y = jnp.maximum((z - mean) * inv * g_ref[...] + bb_ref[...], 0.0)
      for k in range(CHUNKS):
        o_ref[k] = y[:, k * 128:(k + 1) * 128]

  return pl.pallas_call(
      body,
      grid=(2, NB),
      in_specs=[
          pl.BlockSpec((BR, 128), lambda p, i: (i, 0)),
          pl.BlockSpec((NC, BR, 128), lambda p, i: (0, i, 0)),
          pl.BlockSpec((128, HDIM), lambda p, i: (0, 0)),
          pl.BlockSpec((1, HDIM), lambda p, i: (0, 0)),
          pl.BlockSpec((1, HDIM), lambda p, i: (0, 0)),
          pl.BlockSpec((1, HDIM), lambda p, i: (0, 0)),
      ],
      out_specs=pl.BlockSpec((CHUNKS, BR, 128), lambda p, i: (0, i, 0)),
      out_shape=jax.ShapeDtypeStruct((CHUNKS, NPAD, 128), F32),
      scratch_shapes=[pltpu.VMEM((1, HDIM), F32), pltpu.VMEM((1, HDIM), F32)],
  )(hpad, s1p, Wp, b2, g2, bb2)


def _tc_bn2_qemb(x2r, s2r, s1p, g2, bb2):
  NB, BR = 8, NPAD // 8

  def body(x_ref, s_ref, c_ref, g_ref, bb_ref, q_ref, ssum, ssq, qacc):
    p = pl.program_id(0)
    i = pl.program_id(1)
    cnt = (c_ref[0, :, 20] + c_ref[1, :, 20])[:, None]
    rden = 1.0 / jnp.maximum(cnt, 1.0)
    rid = i * BR + lax.broadcasted_iota(jnp.int32, (BR, 1), 0)
    msk = (rid < N).astype(F32)

    @pl.when(jnp.logical_and(p == 0, i == 0))
    def _():
      ssum[...] = jnp.zeros_like(ssum)
      ssq[...] = jnp.zeros_like(ssq)
      qacc[...] = jnp.zeros_like(qacc)

    def _z(k):
      return x_ref[k] + s_ref[k] * rden

    @pl.when(p == 0)
    def _():
      for k in range(CHUNKS):
        sl = pl.ds(k * 128, 128)
        z = _z(k)
        zm = z * msk
        ssum[0, sl] += jnp.sum(zm, 0)
        ssq[0, sl] += jnp.sum(zm * z, 0)

    @pl.when(p == 1)
    def _():
      for k in range(CHUNKS):
        sl = pl.ds(k * 128, 128)
        z = _z(k)
        mean = ssum[0, sl] / N
        var = ssq[0, sl] / N - mean * mean
        inv = lax.rsqrt(var + 1e-5)
        y = jnp.maximum((z - mean) * inv * g_ref[0, sl] + bb_ref[0, sl], 0.0)
        qacc[0, sl] += jnp.sum(y * msk, 0)

      @pl.when(i == NB - 1)
      def _():
        q_ref[...] = qacc[...] / N

  return pl.pallas_call(
      body,
      grid=(2, NB),
      in_specs=[
          pl.BlockSpec((CHUNKS, BR, 128), lambda p, i: (0, i, 0)),
          pl.BlockSpec((CHUNKS, BR, 128), lambda p, i: (0, i, 0)),
          pl.BlockSpec((NC, BR, 128), lambda p, i: (0, i, 0)),
          pl.BlockSpec((1, HDIM), lambda p, i: (0, 0)),
          pl.BlockSpec((1, HDIM), lambda p, i: (0, 0)),
      ],
      out_specs=pl.BlockSpec((1, HDIM), lambda p, i: (0, 0)),
      out_shape=jax.ShapeDtypeStruct((1, HDIM), F32),
      scratch_shapes=[pltpu.VMEM((1, HDIM), F32), pltpu.VMEM((1, HDIM), F32),
                      pltpu.VMEM((1, HDIM), F32)],
  )(x2r, s2r, s1p, g2, bb2)


def _tc_cand(gemb, qemb, Wtop, Wbot, bfc2, g3, b3, w2row, bb2sc):
  NB, BR = 8, G // 8

  def body(ge_ref, q_ref, wt_ref, wb_ref, bf_ref, g_ref, bb_ref, w2_ref,
           b2_ref, o_ref, yscr, ssum, ssq):
    p = pl.program_id(0)
    i = pl.program_id(1)

    @pl.when(jnp.logical_and(p == 0, i == 0))
    def _():
      ssum[...] = jnp.zeros_like(ssum)
      ssq[...] = jnp.zeros_like(ssq)

    @pl.when(p == 0)
    def _():
      c0 = jnp.dot(q_ref[...], wt_ref[...], preferred_element_type=F32)
      yb = jnp.dot(ge_ref[...], wb_ref[...], preferred_element_type=F32)
      yb = yb + c0 + bf_ref[...]
      yscr[pl.ds(i * BR, BR), :] = yb
      ssum[...] += jnp.sum(yb, 0, keepdims=True)
      ssq[...] += jnp.sum(yb * yb, 0, keepdims=True)

    @pl.when(p == 1)
    def _():
      mean = ssum[...] / G
      var = ssq[...] / G - mean * mean
      inv = lax.rsqrt(var + 1e-5)
      yb = yscr[pl.ds(i * BR, BR), :]
      y = jnp.maximum((yb - mean) * inv * g_ref[...] + bb_ref[...], 0.0)
      logit = jnp.sum(y * w2_ref[...], axis=1, keepdims=True) + b2_ref[...]
      prob = 1.0 / (1.0 + jnp.exp(-logit))
      o_ref[...] = prob

  return pl.pallas_call(
      body,
      grid=(2, NB),
      in_specs=[
          pl.BlockSpec((BR, HDIM), lambda p, i: (i, 0)),
          pl.BlockSpec((1, HDIM), lambda p, i: (0, 0)),
          pl.BlockSpec((HDIM, 128), lambda p, i: (0, 0)),
          pl.BlockSpec((HDIM, 128), lambda p, i: (0, 0)),
          pl.BlockSpec((1, 128), lambda p, i: (0, 0)),
          pl.BlockSpec((1, 128), lambda p, i: (0, 0)),
          pl.BlockSpec((1, 128), lambda p, i: (0, 0)),
          pl.BlockSpec((1, 128), lambda p, i: (0, 0)),
          pl.BlockSpec((1, 1), lambda p, i: (0, 0)),
      ],
      out_specs=pl.BlockSpec((BR, 1), lambda p, i: (i, 0)),
      out_shape=jax.ShapeDtypeStruct((G, 1), F32),
      scratch_shapes=[pltpu.VMEM((G, 128), F32), pltpu.VMEM((1, 128), F32),
                      pltpu.VMEM((1, 128), F32)],
  )(gemb, qemb, Wtop, Wbot, bfc2, g3, b3, w2row, bb2sc)


def kernel(h, edge_index, allDBGEmb, gPos,
           W_init, b_init, bn1_scale, bn1_bias, bn2_scale, bn2_bias,
           W_fc, b_fc, bn3_scale, bn3_bias, W_fc2, b_fc2):
  i32 = jnp.int32
  hpad = (jnp.zeros((NPAD, 128), F32).at[:N, :FIN].set(h)
          .at[:, FIN].set(1.0))
  Wp = jnp.zeros((128, HDIM), F32).at[:FIN].set(W_init)
  padv = N + (jnp.arange(EPAD - E, dtype=i32) % (NPAD - N))
  epad = jnp.concatenate(
      [edge_index.astype(i32), jnp.stack([padv, padv])], axis=1)
  src, dst = epad[0], epad[1]
  esrc1 = src.reshape(NW, 40, 128)
  edst1 = dst.reshape(NW, 40, 128)
  src_t = src.reshape(NS, 80, 128)
  edst2 = dst.reshape(NS, 80, 128)
  offs = (jnp.arange(CHUNKS, dtype=i32) * NPAD).reshape(NC, CPC, 1, 1, 1)
  esrc2 = src_t[None, None] + offs
  z128 = jnp.zeros((NPAD, 128), F32)

  gemb = _sc_gather(allDBGEmb, gPos)
  s1p = _sc_conv1(esrc1, edst1, hpad, z128)
  x2r = _tc_fc_bn1(hpad, s1p, Wp,
                   b_init.reshape(1, HDIM), bn1_scale.reshape(1, HDIM),
                   bn1_bias.reshape(1, HDIM))
  s2f = _sc_conv2(esrc2, edst2, x2r.reshape(CHUNKS * NPAD, 128), z128)
  qemb = _tc_bn2_qemb(x2r, s2f.reshape(CHUNKS, NPAD, 128), s1p,
                      bn2_scale.reshape(1, HDIM), bn2_bias.reshape(1, HDIM))
  probs = _tc_cand(gemb, qemb,
                   W_fc[:HDIM], W_fc[HDIM:], b_fc.reshape(1, 128),
                   bn3_scale.reshape(1, 128), bn3_bias.reshape(1, 128),
                   W_fc2.reshape(1, 128), b_fc2.reshape(1, 1))
  return probs.reshape(-1)


# speedup vs baseline: 2.5330x; 1.0396x over previous
import functools

import jax
import jax.numpy as jnp
from jax import lax
from jax.experimental import pallas as pl
from jax.experimental.pallas import tpu as pltpu
from jax.experimental.pallas import tpu_sc as plsc

N = 10000
E = 160000
V = 100000
G = 4096
FIN = 20
HDIM = 1024

NPAD = 10240
EPAD = 163840
NC, NS, NW = 2, 16, 32
CHUNKS = 8
CPC = CHUNKS // NC
RPT = NPAD // NS
F32 = jnp.float32

_MESH = plsc.VectorSubcoreMesh(
    core_axis_name="c", subcore_axis_name="s", num_cores=NC, num_subcores=NS)


def _edge_scan(table, idx_s, idx_d, bufa, bufb, sema, semb, acc, ngroups):
  npair = ngroups // 2

  def _scat(buf, j):
    pltpu.sync_copy(buf, acc.at[idx_d.at[j]], add=True)

  pltpu.async_copy(table.at[idx_s.at[0]], bufa, sema)

  def body(t, carry):
    pltpu.async_copy(table.at[idx_s.at[2 * t + 1]], bufb, semb)
    pltpu.make_async_copy(table.at[idx_s.at[0]], bufa, sema).wait()
    _scat(bufa, 2 * t)
    pltpu.async_copy(table.at[idx_s.at[2 * t + 2]], bufa, sema)
    pltpu.make_async_copy(table.at[idx_s.at[0]], bufb, semb).wait()
    _scat(bufb, 2 * t + 1)
    return carry

  lax.fori_loop(0, npair - 1, body, 0)
  pltpu.async_copy(table.at[idx_s.at[ngroups - 1]], bufb, semb)
  pltpu.make_async_copy(table.at[idx_s.at[0]], bufa, sema).wait()
  _scat(bufa, ngroups - 2)
  pltpu.make_async_copy(table.at[idx_s.at[0]], bufb, semb).wait()
  _scat(bufb, ngroups - 1)


def _sc_conv1(esrc1, edst1, hpad, z32):
  @functools.partial(
      pl.kernel,
      out_type=jax.ShapeDtypeStruct((NC, NPAD, 32), F32),
      mesh=_MESH,
      compiler_params=pltpu.CompilerParams(use_tc_tiling_on_sc=False),
      scratch_types=[
          pltpu.VMEM((40, 128), jnp.int32),
          pltpu.VMEM((40, 128), jnp.int32),
          pltpu.VMEM((128, 32), F32),
          pltpu.VMEM((128, 32), F32),
          pltpu.VMEM_SHARED((NPAD, 32), F32),
          pltpu.SemaphoreType.DMA,
          pltpu.SemaphoreType.DMA,
      ],
  )
  def body(esrc_r, edst_r, hpad_r, z_r, s1_out,
           idx_s, idx_d, bufa, bufb, acc, sema, semb):
    cid = lax.axis_index("c")
    sid = lax.axis_index("s")
    wid = sid * NC + cid
    r0 = sid * RPT
    pltpu.sync_copy(z_r.at[pl.ds(r0, RPT)], acc.at[pl.ds(r0, RPT)])
    pltpu.sync_copy(esrc_r.at[wid], idx_s)
    pltpu.sync_copy(edst_r.at[wid], idx_d)
    plsc.subcore_barrier()
    _edge_scan(hpad_r, idx_s, idx_d, bufa, bufb, sema, semb, acc, 40)
    plsc.subcore_barrier()
    pltpu.sync_copy(acc.at[pl.ds(r0, RPT)], s1_out.at[cid, pl.ds(r0, RPT)])

  return body(esrc1, edst1, hpad, z32)


def _sc_conv2(esrc2, edst2, x2f, z128):
  @functools.partial(
      pl.kernel,
      out_type=jax.ShapeDtypeStruct((CHUNKS * NPAD, 128), F32),
      mesh=_MESH,
      scratch_types=[
          pltpu.VMEM((40, 128), jnp.int32),
          pltpu.VMEM((40, 128), jnp.int32),
          pltpu.VMEM((128, 128), F32),
          pltpu.VMEM((128, 128), F32),
          pltpu.VMEM_SHARED((NPAD, 128), F32),
          pltpu.SemaphoreType.DMA,
          pltpu.SemaphoreType.DMA,
      ],
  )
  def body(esrc_r, edst_r, x2f_r, z_r, s2_out,
           idx_s, idx_d, bufa, bufb, acc, sema, semb):
    cid = lax.axis_index("c")
    sid = lax.axis_index("s")
    r0 = sid * RPT
    for k in range(CPC):
      pltpu.sync_copy(z_r.at[pl.ds(r0, RPT)], acc.at[pl.ds(r0, RPT)])
      plsc.subcore_barrier()
      for half in range(2):
        pltpu.sync_copy(esrc_r.at[cid, k, sid, pl.ds(half * 40, 40)], idx_s)
        pltpu.sync_copy(edst_r.at[sid, pl.ds(half * 40, 40)], idx_d)
        _edge_scan(x2f_r, idx_s, idx_d, bufa, bufb, sema, semb, acc, 40)
      plsc.subcore_barrier()
      off = (cid * CPC + k) * NPAD + r0
      pltpu.sync_copy(acc.at[pl.ds(r0, RPT)], s2_out.at[pl.ds(off, RPT)])

  return body(esrc2, edst2, x2f, z128)


def _sc_gather(table, gpos):
  rows_per_w = G // NW

  @functools.partial(
      pl.kernel,
      out_type=jax.ShapeDtypeStruct((G, HDIM), F32),
      mesh=_MESH,
      scratch_types=[
          pltpu.VMEM((rows_per_w,), jnp.int32),
          pltpu.VMEM((rows_per_w // 2, HDIM), F32),
          pltpu.SemaphoreType.DMA,
      ],
  )
  def body(table_r, gpos_r, out_r, idx_v, rows, sem):
    cid = lax.axis_index("c")
    sid = lax.axis_index("s")
    wid = sid * NC + cid
    pltpu.sync_copy(gpos_r.at[pl.ds(wid * rows_per_w, rows_per_w)], idx_v)
    for half in range(2):
      hh = rows_per_w // 2
      pltpu.async_copy(
          table_r.at[idx_v.at[pl.ds(half * hh, hh)]], rows, sem).wait()
      pltpu.sync_copy(rows, out_r.at[pl.ds(wid * rows_per_w + half * hh, hh)])

  return body(table, gpos)


def _tc_fc_bn1(hpad, s1p, Wp, b2, g2, bb2):
  NB, BR = 8, NPAD // 8

  def body(h_ref, s1_ref, w_ref, b_ref, g_ref, bb_ref, o_ref,
           ssum, ssq):
    p = pl.program_id(0)
    i = pl.program_id(1)
    s1 = s1_ref[0] + s1_ref[1]
    cnt = s1[:, 20][:, None]
    u = h_ref[...] + s1 / jnp.maximum(cnt, 1.0)
    z = jnp.dot(u, w_ref[...], preferred_element_type=F32)
    z = z + b_ref[...] * (1.0 + (cnt > 0).astype(F32))
    rid = i * BR + lax.broadcasted_iota(jnp.int32, (BR, 1), 0)
    msk = (rid < N).astype(F32)

    @pl.when(jnp.logical_and(p == 0, i == 0))
    def _():
      ssum[...] = jnp.zeros_like(ssum)
      ssq[...] = jnp.zeros_like(ssq)

    @pl.when(p == 0)
    def _():
      zm = z * msk
      ssum[...] += jnp.sum(zm, 0, keepdims=True)
      ssq[...] += jnp.sum(zm * z, 0, keepdims=True)

    @pl.when(p == 1)
    def _():
      mean = ssum[...] / N
      var = ssq[...] / N - mean * mean
      inv = lax.rsqrt(var + 1e-5)
      y = jnp.maximum((z - mean) * inv * g_ref[...] + bb_ref[...], 0.0)
      for k in range(CHUNKS):
        o_ref[k] = y[:, k * 128:(k + 1) * 128]

  return pl.pallas_call(
      body,
      grid=(2, NB),
      in_specs=[
          pl.BlockSpec((BR, 32), lambda p, i: (i, 0)),
          pl.BlockSpec((NC, BR, 32), lambda p, i: (0, i, 0)),
          pl.BlockSpec((32, HDIM), lambda p, i: (0, 0)),
          pl.BlockSpec((1, HDIM), lambda p, i: (0, 0)),
          pl.BlockSpec((1, HDIM), lambda p, i: (0, 0)),
          pl.BlockSpec((1, HDIM), lambda p, i: (0, 0)),
      ],
      out_specs=pl.BlockSpec((CHUNKS, BR, 128), lambda p, i: (0, i * p, 0)),
      out_shape=jax.ShapeDtypeStruct((CHUNKS, NPAD, 128), F32),
      scratch_shapes=[pltpu.VMEM((1, HDIM), F32), pltpu.VMEM((1, HDIM), F32)],
  )(hpad, s1p, Wp, b2, g2, bb2)


def _tc_bn2_qemb(x2r, s2r, s1p, g2, bb2):
  NB, BR = 8, NPAD // 8

  def body(x_ref, s_ref, c_ref, g_ref, bb_ref, q_ref, ssum, ssq, qacc):
    p = pl.program_id(0)
    i = pl.program_id(1)
    cnt = (c_ref[0, :, 20] + c_ref[1, :, 20])[:, None]
    rden = 1.0 / jnp.maximum(cnt, 1.0)
    rid = i * BR + lax.broadcasted_iota(jnp.int32, (BR, 1), 0)
    msk = (rid < N).astype(F32)

    @pl.when(jnp.logical_and(p == 0, i == 0))
    def _():
      ssum[...] = jnp.zeros_like(ssum)
      ssq[...] = jnp.zeros_like(ssq)
      qacc[...] = jnp.zeros_like(qacc)

    def _z(k):
      return x_ref[k] + s_ref[k] * rden

    @pl.when(p == 0)
    def _():
      for k in range(CHUNKS):
        sl = pl.ds(k * 128, 128)
        z = _z(k)
        zm = z * msk
        ssum[0, sl] += jnp.sum(zm, 0)
        ssq[0, sl] += jnp.sum(zm * z, 0)

    @pl.when(p == 1)
    def _():
      for k in range(CHUNKS):
        sl = pl.ds(k * 128, 128)
        z = _z(k)
        mean = ssum[0, sl] / N
        var = ssq[0, sl] / N - mean * mean
        inv = lax.rsqrt(var + 1e-5)
        y = jnp.maximum((z - mean) * inv * g_ref[0, sl] + bb_ref[0, sl], 0.0)
        qacc[0, sl] += jnp.sum(y * msk, 0)

      @pl.when(i == NB - 1)
      def _():
        q_ref[...] = qacc[...] / N

  return pl.pallas_call(
      body,
      grid=(2, NB),
      in_specs=[
          pl.BlockSpec((CHUNKS, BR, 128), lambda p, i: (0, i, 0)),
          pl.BlockSpec((CHUNKS, BR, 128), lambda p, i: (0, i, 0)),
          pl.BlockSpec((NC, BR, 32), lambda p, i: (0, i, 0)),
          pl.BlockSpec((1, HDIM), lambda p, i: (0, 0)),
          pl.BlockSpec((1, HDIM), lambda p, i: (0, 0)),
      ],
      out_specs=pl.BlockSpec((1, HDIM), lambda p, i: (0, 0)),
      out_shape=jax.ShapeDtypeStruct((1, HDIM), F32),
      scratch_shapes=[pltpu.VMEM((1, HDIM), F32), pltpu.VMEM((1, HDIM), F32),
                      pltpu.VMEM((1, HDIM), F32)],
  )(x2r, s2r, s1p, g2, bb2)


def _tc_cand(gemb, qemb, Wtop, Wbot, bfc2, g3, b3, w2row, bb2sc):
  NB, BR = 8, G // 8

  def body(ge_ref, q_ref, wt_ref, wb_ref, bf_ref, g_ref, bb_ref, w2_ref,
           b2_ref, o_ref, yscr, ssum, ssq):
    p = pl.program_id(0)
    i = pl.program_id(1)

    @pl.when(jnp.logical_and(p == 0, i == 0))
    def _():
      ssum[...] = jnp.zeros_like(ssum)
      ssq[...] = jnp.zeros_like(ssq)

    @pl.when(p == 0)
    def _():
      c0 = jnp.dot(q_ref[...], wt_ref[...], preferred_element_type=F32)
      yb = jnp.dot(ge_ref[...], wb_ref[...], preferred_element_type=F32)
      yb = yb + c0 + bf_ref[...]
      yscr[pl.ds(i * BR, BR), :] = yb
      ssum[...] += jnp.sum(yb, 0, keepdims=True)
      ssq[...] += jnp.sum(yb * yb, 0, keepdims=True)

    @pl.when(p == 1)
    def _():
      mean = ssum[...] / G
      var = ssq[...] / G - mean * mean
      inv = lax.rsqrt(var + 1e-5)
      yb = yscr[pl.ds(i * BR, BR), :]
      y = jnp.maximum((yb - mean) * inv * g_ref[...] + bb_ref[...], 0.0)
      logit = jnp.sum(y * w2_ref[...], axis=1, keepdims=True) + b2_ref[...]
      prob = 1.0 / (1.0 + jnp.exp(-logit))
      o_ref[...] = prob

  return pl.pallas_call(
      body,
      grid=(2, NB),
      in_specs=[
          pl.BlockSpec((BR, HDIM), lambda p, i: (i, 0)),
          pl.BlockSpec((1, HDIM), lambda p, i: (0, 0)),
          pl.BlockSpec((HDIM, 128), lambda p, i: (0, 0)),
          pl.BlockSpec((HDIM, 128), lambda p, i: (0, 0)),
          pl.BlockSpec((1, 128), lambda p, i: (0, 0)),
          pl.BlockSpec((1, 128), lambda p, i: (0, 0)),
          pl.BlockSpec((1, 128), lambda p, i: (0, 0)),
          pl.BlockSpec((1, 128), lambda p, i: (0, 0)),
          pl.BlockSpec((1, 1), lambda p, i: (0, 0)),
      ],
      out_specs=pl.BlockSpec((BR, 1), lambda p, i: (i, 0)),
      out_shape=jax.ShapeDtypeStruct((G, 1), F32),
      scratch_shapes=[pltpu.VMEM((G, 128), F32), pltpu.VMEM((1, 128), F32),
                      pltpu.VMEM((1, 128), F32)],
  )(gemb, qemb, Wtop, Wbot, bfc2, g3, b3, w2row, bb2sc)


def kernel(h, edge_index, allDBGEmb, gPos,
           W_init, b_init, bn1_scale, bn1_bias, bn2_scale, bn2_bias,
           W_fc, b_fc, bn3_scale, bn3_bias, W_fc2, b_fc2):
  i32 = jnp.int32
  hpad = (jnp.zeros((NPAD, 32), F32).at[:N, :FIN].set(h)
          .at[:, FIN].set(1.0))
  Wp = jnp.zeros((32, HDIM), F32).at[:FIN].set(W_init)
  padv = N + (jnp.arange(EPAD - E, dtype=i32) % (NPAD - N))
  epad = jnp.concatenate(
      [edge_index.astype(i32), jnp.stack([padv, padv])], axis=1)
  src, dst = epad[0], epad[1]
  esrc1 = src.reshape(NW, 40, 128)
  edst1 = dst.reshape(NW, 40, 128)
  src_t = src.reshape(NS, 80, 128)
  edst2 = dst.reshape(NS, 80, 128)
  offs = (jnp.arange(CHUNKS, dtype=i32) * NPAD).reshape(NC, CPC, 1, 1, 1)
  esrc2 = src_t[None, None] + offs
  z128 = jnp.zeros((NPAD, 128), F32)
  z32 = jnp.zeros((NPAD, 32), F32)

  s1p = _sc_conv1(esrc1, edst1, hpad, z32)
  gemb = _sc_gather(allDBGEmb, gPos)
  x2r = _tc_fc_bn1(hpad, s1p, Wp,
                   b_init.reshape(1, HDIM), bn1_scale.reshape(1, HDIM),
                   bn1_bias.reshape(1, HDIM))
  s2f = _sc_conv2(esrc2, edst2, x2r.reshape(CHUNKS * NPAD, 128), z128)
  qemb = _tc_bn2_qemb(x2r, s2f.reshape(CHUNKS, NPAD, 128), s1p,
                      bn2_scale.reshape(1, HDIM), bn2_bias.reshape(1, HDIM))
  probs = _tc_cand(gemb, qemb,
                   W_fc[:HDIM], W_fc[HDIM:], b_fc.reshape(1, 128),
                   bn3_scale.reshape(1, 128), bn3_bias.reshape(1, 128),
                   W_fc2.reshape(1, 128), b_fc2.reshape(1, 1))
  return probs.reshape(-1)


# speedup vs baseline: 2.6022x; 1.0273x over previous
import functools

import jax
import jax.numpy as jnp
from jax import lax
from jax.experimental import pallas as pl
from jax.experimental.pallas import tpu as pltpu
from jax.experimental.pallas import tpu_sc as plsc

N = 10000
E = 160000
V = 100000
G = 4096
FIN = 20
HDIM = 1024

NPAD = 10240
EPAD = 163840
NC, NS, NW = 2, 16, 32
CHUNKS = 8
CPC = CHUNKS // NC
RPT = NPAD // NS
F32 = jnp.float32

_MESH = plsc.VectorSubcoreMesh(
    core_axis_name="c", subcore_axis_name="s", num_cores=NC, num_subcores=NS)


def _edge_scan(table, idx_s, idx_d, bufa, bufb, sema, semb, acc, ngroups):
  npair = ngroups // 2

  def _scat(buf, j):
    pltpu.sync_copy(buf, acc.at[idx_d.at[j]], add=True)

  pltpu.async_copy(table.at[idx_s.at[0]], bufa, sema)

  def body(t, carry):
    pltpu.async_copy(table.at[idx_s.at[2 * t + 1]], bufb, semb)
    pltpu.make_async_copy(table.at[idx_s.at[0]], bufa, sema).wait()
    _scat(bufa, 2 * t)
    pltpu.async_copy(table.at[idx_s.at[2 * t + 2]], bufa, sema)
    pltpu.make_async_copy(table.at[idx_s.at[0]], bufb, semb).wait()
    _scat(bufb, 2 * t + 1)
    return carry

  lax.fori_loop(0, npair - 1, body, 0)
  pltpu.async_copy(table.at[idx_s.at[ngroups - 1]], bufb, semb)
  pltpu.make_async_copy(table.at[idx_s.at[0]], bufa, sema).wait()
  _scat(bufa, ngroups - 2)
  pltpu.make_async_copy(table.at[idx_s.at[0]], bufb, semb).wait()
  _scat(bufb, ngroups - 1)


def _sc_conv1(esrc1, edst1, hpad, z32):
  @functools.partial(
      pl.kernel,
      out_type=jax.ShapeDtypeStruct((NC, NPAD, 32), F32),
      mesh=_MESH,
      compiler_params=pltpu.CompilerParams(use_tc_tiling_on_sc=False),
      scratch_types=[
          pltpu.VMEM((40, 128), jnp.int32),
          pltpu.VMEM((40, 128), jnp.int32),
          pltpu.VMEM((128, 32), F32),
          pltpu.VMEM((128, 32), F32),
          pltpu.VMEM_SHARED((NPAD, 32), F32),
          pltpu.SemaphoreType.DMA,
          pltpu.SemaphoreType.DMA,
      ],
  )
  def body(esrc_r, edst_r, hpad_r, z_r, s1_out,
           idx_s, idx_d, bufa, bufb, acc, sema, semb):
    cid = lax.axis_index("c")
    sid = lax.axis_index("s")
    wid = sid * NC + cid
    r0 = sid * RPT
    pltpu.sync_copy(z_r.at[pl.ds(r0, RPT)], acc.at[pl.ds(r0, RPT)])
    pltpu.sync_copy(esrc_r.at[wid], idx_s)
    pltpu.sync_copy(edst_r.at[wid], idx_d)
    plsc.subcore_barrier()
    _edge_scan(hpad_r, idx_s, idx_d, bufa, bufb, sema, semb, acc, 40)
    plsc.subcore_barrier()
    pltpu.sync_copy(acc.at[pl.ds(r0, RPT)], s1_out.at[cid, pl.ds(r0, RPT)])

  return body(esrc1, edst1, hpad, z32)


def _sc_conv2(esrc2, edst2, x2f, z128):
  @functools.partial(
      pl.kernel,
      out_type=jax.ShapeDtypeStruct((CHUNKS * NPAD, 128), F32),
      mesh=_MESH,
      scratch_types=[
          pltpu.VMEM((40, 128), jnp.int32),
          pltpu.VMEM((40, 128), jnp.int32),
          pltpu.VMEM((128, 128), F32),
          pltpu.VMEM((128, 128), F32),
          pltpu.VMEM_SHARED((NPAD, 128), F32),
          pltpu.SemaphoreType.DMA,
          pltpu.SemaphoreType.DMA,
      ],
  )
  def body(esrc_r, edst_r, x2f_r, z_r, s2_out,
           idx_s, idx_d, bufa, bufb, acc, sema, semb):
    cid = lax.axis_index("c")
    sid = lax.axis_index("s")
    r0 = sid * RPT
    for k in range(CPC):
      pltpu.sync_copy(z_r.at[pl.ds(r0, RPT)], acc.at[pl.ds(r0, RPT)])
      plsc.subcore_barrier()
      for half in range(2):
        pltpu.sync_copy(esrc_r.at[cid, k, sid, pl.ds(half * 40, 40)], idx_s)
        pltpu.sync_copy(edst_r.at[sid, pl.ds(half * 40, 40)], idx_d)
        _edge_scan(x2f_r, idx_s, idx_d, bufa, bufb, sema, semb, acc, 40)
      plsc.subcore_barrier()
      off = (cid * CPC + k) * NPAD + r0
      pltpu.sync_copy(acc.at[pl.ds(r0, RPT)], s2_out.at[pl.ds(off, RPT)])

  return body(esrc2, edst2, x2f, z128)


def _sc_gather(table, gpos):
  rows_per_w = G // NW

  @functools.partial(
      pl.kernel,
      out_type=jax.ShapeDtypeStruct((G, HDIM), F32),
      mesh=_MESH,
      scratch_types=[
          pltpu.VMEM((rows_per_w,), jnp.int32),
          pltpu.VMEM((rows_per_w // 2, HDIM), F32),
          pltpu.SemaphoreType.DMA,
      ],
  )
  def body(table_r, gpos_r, out_r, idx_v, rows, sem):
    cid = lax.axis_index("c")
    sid = lax.axis_index("s")
    wid = sid * NC + cid
    pltpu.sync_copy(gpos_r.at[pl.ds(wid * rows_per_w, rows_per_w)], idx_v)
    for half in range(2):
      hh = rows_per_w // 2
      pltpu.async_copy(
          table_r.at[idx_v.at[pl.ds(half * hh, hh)]], rows, sem).wait()
      pltpu.sync_copy(rows, out_r.at[pl.ds(wid * rows_per_w + half * hh, hh)])

  return body(table, gpos)


def _tc_fc_bn1(hpad, s1p, Wp, b2, g2, bb2):
  NB, BR = 8, NPAD // 8

  def body(h_ref, s1_ref, w_ref, b_ref, g_ref, bb_ref, o_ref,
           ssum, ssq):
    p = pl.program_id(0)
    i = pl.program_id(1)
    s1 = s1_ref[0] + s1_ref[1]
    cnt = s1[:, 20][:, None]
    u = h_ref[...] + s1 / jnp.maximum(cnt, 1.0)
    z = jnp.dot(u, w_ref[...], preferred_element_type=F32)
    z = z + b_ref[...] * (1.0 + (cnt > 0).astype(F32))
    rid = i * BR + lax.broadcasted_iota(jnp.int32, (BR, 1), 0)
    msk = (rid < N).astype(F32)

    @pl.when(jnp.logical_and(p == 0, i == 0))
    def _():
      ssum[...] = jnp.zeros_like(ssum)
      ssq[...] = jnp.zeros_like(ssq)

    @pl.when(p == 0)
    def _():
      zm = z * msk
      ssum[...] += jnp.sum(zm, 0, keepdims=True)
      ssq[...] += jnp.sum(zm * z, 0, keepdims=True)

    @pl.when(p == 1)
    def _():
      mean = ssum[...] / N
      var = ssq[...] / N - mean * mean
      inv = lax.rsqrt(var + 1e-5)
      y = jnp.maximum((z - mean) * inv * g_ref[...] + bb_ref[...], 0.0)
      for k in range(CHUNKS):
        o_ref[k] = y[:, k * 128:(k + 1) * 128]

  return pl.pallas_call(
      body,
      grid=(2, NB),
      in_specs=[
          pl.BlockSpec((BR, 32), lambda p, i: (i, 0)),
          pl.BlockSpec((NC, BR, 32), lambda p, i: (0, i, 0)),
          pl.BlockSpec((32, HDIM), lambda p, i: (0, 0)),
          pl.BlockSpec((1, HDIM), lambda p, i: (0, 0)),
          pl.BlockSpec((1, HDIM), lambda p, i: (0, 0)),
          pl.BlockSpec((1, HDIM), lambda p, i: (0, 0)),
      ],
      out_specs=pl.BlockSpec((CHUNKS, BR, 128), lambda p, i: (0, i * p, 0)),
      out_shape=jax.ShapeDtypeStruct((CHUNKS, NPAD, 128), F32),
      scratch_shapes=[pltpu.VMEM((1, HDIM), F32), pltpu.VMEM((1, HDIM), F32)],
  )(hpad, s1p, Wp, b2, g2, bb2)


def _tc_bn2_cand(x2r, s2r, s1p, g2, bb2,
                 gemb, Wtop, Wbot, bfc2, g3, b3, w2row, bb2sc):
  NB, BR = 8, NPAD // 8

  def body(x_ref, s_ref, c_ref, g_ref, bb_ref,
           ge_ref, wt_ref, wb_ref, bf_ref, g3_ref, b3_ref, w2_ref, b2_ref,
           o_ref, ssum, ssq, qacc):
    p = pl.program_id(0)
    i = pl.program_id(1)
    cnt = (c_ref[0, :, 20] + c_ref[1, :, 20])[:, None]
    rden = 1.0 / jnp.maximum(cnt, 1.0)
    rid = i * BR + lax.broadcasted_iota(jnp.int32, (BR, 1), 0)
    msk = (rid < N).astype(F32)

    @pl.when(jnp.logical_and(p == 0, i == 0))
    def _():
      ssum[...] = jnp.zeros_like(ssum)
      ssq[...] = jnp.zeros_like(ssq)
      qacc[...] = jnp.zeros_like(qacc)

    def _z(k):
      return x_ref[k] + s_ref[k] * rden

    @pl.when(p == 0)
    def _():
      for k in range(CHUNKS):
        sl = pl.ds(k * 128, 128)
        z = _z(k)
        zm = z * msk
        ssum[0, sl] += jnp.sum(zm, 0)
        ssq[0, sl] += jnp.sum(zm * z, 0)

    @pl.when(p == 1)
    def _():
      for k in range(CHUNKS):
        sl = pl.ds(k * 128, 128)
        z = _z(k)
        mean = ssum[0, sl] / N
        var = ssq[0, sl] / N - mean * mean
        inv = lax.rsqrt(var + 1e-5)
        y = jnp.maximum((z - mean) * inv * g_ref[0, sl] + bb_ref[0, sl], 0.0)
        qacc[0, sl] += jnp.sum(y * msk, 0)

      @pl.when(i == NB - 1)
      def _():
        qemb = qacc[...] / N
        c0 = jnp.dot(qemb, wt_ref[...], preferred_element_type=F32)
        yb = jnp.dot(ge_ref[...], wb_ref[...], preferred_element_type=F32)
        yb = yb + c0 + bf_ref[...]
        m3 = jnp.mean(yb, 0, keepdims=True)
        v3 = jnp.mean(yb * yb, 0, keepdims=True) - m3 * m3
        inv3 = lax.rsqrt(v3 + 1e-5)
        y3 = jnp.maximum((yb - m3) * inv3 * g3_ref[...] + b3_ref[...], 0.0)
        logit = jnp.sum(y3 * w2_ref[...], axis=1, keepdims=True) + b2_ref[...]
        o_ref[...] = 1.0 / (1.0 + jnp.exp(-logit))

  return pl.pallas_call(
      body,
      grid=(2, NB),
      in_specs=[
          pl.BlockSpec((CHUNKS, BR, 128), lambda p, i: (0, i, 0)),
          pl.BlockSpec((CHUNKS, BR, 128), lambda p, i: (0, i, 0)),
          pl.BlockSpec((NC, BR, 32), lambda p, i: (0, i, 0)),
          pl.BlockSpec((1, HDIM), lambda p, i: (0, 0)),
          pl.BlockSpec((1, HDIM), lambda p, i: (0, 0)),
          pl.BlockSpec((G, HDIM), lambda p, i: (0, 0)),
          pl.BlockSpec((HDIM, 128), lambda p, i: (0, 0)),
          pl.BlockSpec((HDIM, 128), lambda p, i: (0, 0)),
          pl.BlockSpec((1, 128), lambda p, i: (0, 0)),
          pl.BlockSpec((1, 128), lambda p, i: (0, 0)),
          pl.BlockSpec((1, 128), lambda p, i: (0, 0)),
          pl.BlockSpec((1, 128), lambda p, i: (0, 0)),
          pl.BlockSpec((1, 1), lambda p, i: (0, 0)),
      ],
      out_specs=pl.BlockSpec((G, 1), lambda p, i: (0, 0)),
      out_shape=jax.ShapeDtypeStruct((G, 1), F32),
      scratch_shapes=[pltpu.VMEM((1, HDIM), F32), pltpu.VMEM((1, HDIM), F32),
                      pltpu.VMEM((1, HDIM), F32)],
  )(x2r, s2r, s1p, g2, bb2, gemb, Wtop, Wbot, bfc2, g3, b3, w2row, bb2sc)


def kernel(h, edge_index, allDBGEmb, gPos,
           W_init, b_init, bn1_scale, bn1_bias, bn2_scale, bn2_bias,
           W_fc, b_fc, bn3_scale, bn3_bias, W_fc2, b_fc2):
  i32 = jnp.int32
  hpad = (jnp.zeros((NPAD, 32), F32).at[:N, :FIN].set(h)
          .at[:, FIN].set(1.0))
  Wp = jnp.zeros((32, HDIM), F32).at[:FIN].set(W_init)
  padv = N + (jnp.arange(EPAD - E, dtype=i32) % (NPAD - N))
  epad = jnp.concatenate(
      [edge_index.astype(i32), jnp.stack([padv, padv])], axis=1)
  src, dst = epad[0], epad[1]
  esrc1 = src.reshape(NW, 40, 128)
  edst1 = dst.reshape(NW, 40, 128)
  src_t = src.reshape(NS, 80, 128)
  edst2 = dst.reshape(NS, 80, 128)
  offs = (jnp.arange(CHUNKS, dtype=i32) * NPAD).reshape(NC, CPC, 1, 1, 1)
  esrc2 = src_t[None, None] + offs
  z128 = jnp.zeros((NPAD, 128), F32)
  z32 = jnp.zeros((NPAD, 32), F32)

  s1p = _sc_conv1(esrc1, edst1, hpad, z32)
  gemb = _sc_gather(allDBGEmb, gPos)
  x2r = _tc_fc_bn1(hpad, s1p, Wp,
                   b_init.reshape(1, HDIM), bn1_scale.reshape(1, HDIM),
                   bn1_bias.reshape(1, HDIM))
  s2f = _sc_conv2(esrc2, edst2, x2r.reshape(CHUNKS * NPAD, 128), z128)
  probs = _tc_bn2_cand(x2r, s2f.reshape(CHUNKS, NPAD, 128), s1p,
                       bn2_scale.reshape(1, HDIM), bn2_bias.reshape(1, HDIM),
                       gemb, W_fc[:HDIM], W_fc[HDIM:], b_fc.reshape(1, 128),
                       bn3_scale.reshape(1, 128), bn3_bias.reshape(1, 128),
                       W_fc2.reshape(1, 128), b_fc2.reshape(1, 1))
  return probs.reshape(-1)
